# Initial kernel scaffold; baseline (speedup 1.0000x reference)
import jax
import jax.numpy as jnp
from jax.experimental import pallas as pl


def kernel(student_features, teacher_codes, codebook):
    raise NotImplementedError("write your pallas kernel here")


# speedup vs baseline: 2.0721x; 2.0721x over previous
import functools

import jax
import jax.numpy as jnp
from jax import lax
from jax.experimental import pallas as pl
from jax.experimental.pallas import tpu as pltpu
from jax.experimental.pallas import tpu_sc as plsc

_MARGIN = 0.2
_EPS = 1e-6

_NC = 2
_NS = 16
_NW = _NC * _NS

_TCH = 512
_IDXW = 128


def _vsqrt(s):
    i = lax.bitcast_convert_type(s, jnp.int32)
    y = lax.bitcast_convert_type(
        jnp.int32(0x5F3759DF) - lax.shift_right_logical(i, 1), jnp.float32)
    for _ in range(4):
        y = y * (1.5 - 0.5 * s * y * y)
    return s * y


@functools.partial(jax.jit, static_argnums=(4,))
def _sc_triplet(student_features, codes3, rand3, codebook, v_size):
    B, C, L = student_features.shape
    n_chunk = L // _TCH
    rows_per_chunk = _TCH // _IDXW
    groups = _TCH // 16

    mesh = plsc.VectorSubcoreMesh(core_axis_name="c", subcore_axis_name="s")

    @functools.partial(
        pl.kernel,
        mesh=mesh,
        compiler_params=pltpu.CompilerParams(
            needs_layout_passes=False, use_tc_tiling_on_sc=False),
        out_type=jax.ShapeDtypeStruct((_NW, 4, 16), jnp.float32),
        scratch_types=[
            pltpu.VMEM((C, _TCH), jnp.float32),
            pltpu.VMEM((rows_per_chunk, _IDXW), jnp.int32),
            pltpu.VMEM((rows_per_chunk, _IDXW), jnp.int32),
            pltpu.VMEM((rows_per_chunk, _IDXW), jnp.int32),
            pltpu.VMEM((_TCH, C), jnp.float32),
            pltpu.VMEM((_TCH, C), jnp.float32),
            pltpu.VMEM((4, 16), jnp.float32),
            pltpu.SemaphoreType.DMA,
        ],
    )
    def sc_kernel(sf_hbm, codes_hbm, rand_hbm, cb_hbm, out_hbm,
                  f_v, pidx_v, ridx_v, nidx_v, pos_v, neg_v, res_v, sem):
        cid = lax.axis_index("c")
        sid = lax.axis_index("s")
        wid = sid * _NC + cid

        def chunk_body(j, carry):
            acc_l, acc_p, acc_n = carry
            l0 = j * _TCH
            r0 = j * rows_per_chunk
            pltpu.sync_copy(sf_hbm.at[wid, :, pl.ds(l0, _TCH)], f_v)
            pltpu.sync_copy(codes_hbm.at[wid, pl.ds(r0, rows_per_chunk), :],
                            pidx_v)
            pltpu.sync_copy(rand_hbm.at[wid, pl.ds(r0, rows_per_chunk), :],
                            ridx_v)
            for i in range(rows_per_chunk):
                for k in range(_IDXW // 16):
                    r = ridx_v[i, pl.ds(k * 16, 16)]
                    c = pidx_v[i, pl.ds(k * 16, 16)]
                    nidx_v[i, pl.ds(k * 16, 16)] = jnp.where(
                        r == c, lax.rem(r + 1, jnp.int32(v_size)), r)
            copies = []
            for i in range(rows_per_chunk):
                copies.append(pltpu.async_copy(
                    cb_hbm.at[pidx_v.at[i]],
                    pos_v.at[pl.ds(i * _IDXW, _IDXW)], sem))
            for i in range(rows_per_chunk):
                copies.append(pltpu.async_copy(
                    cb_hbm.at[nidx_v.at[i]],
                    neg_v.at[pl.ds(i * _IDXW, _IDXW)], sem))
            for cp in copies:
                cp.wait()

            def group_body(g, gcarry):
                al, ap, an = gcarry
                t0 = g * 16
                rows = t0 + lax.iota(jnp.int32, 16)
                dp2 = [jnp.zeros((16,), jnp.float32) for _ in range(4)]
                dn2 = [jnp.zeros((16,), jnp.float32) for _ in range(4)]
                for c in range(C):
                    col = jnp.full((16,), c, jnp.int32)
                    f = f_v[c, pl.ds(t0, 16)] + _EPS
                    p = plsc.load_gather(pos_v, [rows, col])
                    n = plsc.load_gather(neg_v, [rows, col])
                    dp = f - p
                    dn = f - n
                    dp2[c % 4] = dp2[c % 4] + dp * dp
                    dn2[c % 4] = dn2[c % 4] + dn * dn
                d_pos = _vsqrt((dp2[0] + dp2[1]) + (dp2[2] + dp2[3]))
                d_neg = _vsqrt((dn2[0] + dn2[1]) + (dn2[2] + dn2[3]))
                t = jnp.maximum(d_pos - d_neg + _MARGIN, 0.0)
                return (al + t, ap + d_pos, an + d_neg)

            return lax.fori_loop(0, groups, group_body,
                                 (acc_l, acc_p, acc_n))

        zero = jnp.zeros((16,), jnp.float32)
        acc_l, acc_p, acc_n = lax.fori_loop(0, n_chunk, chunk_body,
                                            (zero, zero, zero))
        res_v[0, :] = acc_l
        res_v[1, :] = acc_p
        res_v[2, :] = acc_n
        res_v[3, :] = jnp.zeros((16,), jnp.float32)
        pltpu.sync_copy(res_v, out_hbm.at[wid])

    return sc_kernel(student_features, codes3, rand3, codebook)


def kernel(student_features, teacher_codes, codebook):
    B, C, L = student_features.shape
    if teacher_codes.ndim == 3:
        teacher_codes = teacher_codes[0]
    V = codebook.shape[0]
    N = B * L
    rand = jax.random.randint(jax.random.key(42), (N,), 0, V)
    codes3 = teacher_codes.reshape(B, L // _IDXW, _IDXW).astype(jnp.int32)
    rand3 = rand.reshape(B, L // _IDXW, _IDXW).astype(jnp.int32)
    part = _sc_triplet(student_features, codes3, rand3, codebook, V)
    sums = part[:, :3, :].sum(axis=(0, 2))
    inv = jnp.float32(1.0 / N)
    return (sums[0] * inv, sums[1] * inv, sums[2] * inv)


# speedup vs baseline: 2.4960x; 1.2046x over previous
import functools

import jax
import jax.numpy as jnp
from jax import lax
from jax.experimental import pallas as pl
from jax.experimental.pallas import tpu as pltpu
from jax.experimental.pallas import tpu_sc as plsc

_MARGIN = 0.2
_EPS = 1e-6

_NC = 2
_NS = 16
_NW = _NC * _NS

_TCH = 512
_IDXW = 128
_CUNROLL = 8


def _vsqrt(s):
    i = lax.bitcast_convert_type(s, jnp.int32)
    y = lax.bitcast_convert_type(
        jnp.int32(0x5F3759DF) - lax.shift_right_logical(i, 1), jnp.float32)
    for _ in range(4):
        y = y * (1.5 - 0.5 * s * y * y)
    return s * y


@functools.partial(jax.jit, static_argnums=(4,))
def _sc_triplet(student_features, codes3, rand3, codebook, v_size):
    B, C, L = student_features.shape
    n_chunk = L // _TCH
    rows_per_chunk = _TCH // _IDXW
    groups = _TCH // 16

    mesh = plsc.VectorSubcoreMesh(core_axis_name="c", subcore_axis_name="s")

    @functools.partial(
        pl.kernel,
        mesh=mesh,
        compiler_params=pltpu.CompilerParams(
            needs_layout_passes=False, use_tc_tiling_on_sc=False),
        out_type=jax.ShapeDtypeStruct((_NW, 4, 16), jnp.float32),
        scratch_types=[
            pltpu.VMEM((C, _TCH), jnp.float32),
            pltpu.VMEM((rows_per_chunk, _IDXW), jnp.int32),
            pltpu.VMEM((rows_per_chunk, _IDXW), jnp.int32),
            pltpu.VMEM((rows_per_chunk, _IDXW), jnp.int32),
            pltpu.VMEM((_TCH, C), jnp.float32),
            pltpu.VMEM((_TCH, C), jnp.float32),
            pltpu.VMEM((4, 16), jnp.float32),
            pltpu.SemaphoreType.DMA,
        ],
    )
    def sc_kernel(sf_hbm, codes_hbm, rand_hbm, cb_hbm, out_hbm,
                  f_v, pidx_v, ridx_v, nidx_v, pos_v, neg_v, res_v, sem):
        cid = lax.axis_index("c")
        sid = lax.axis_index("s")
        wid = sid * _NC + cid

        def chunk_body(j, carry):
            acc_l, acc_p, acc_n = carry
            l0 = j * _TCH
            r0 = j * rows_per_chunk
            pltpu.sync_copy(sf_hbm.at[wid, :, pl.ds(l0, _TCH)], f_v)
            pltpu.sync_copy(codes_hbm.at[wid, pl.ds(r0, rows_per_chunk), :],
                            pidx_v)
            pltpu.sync_copy(rand_hbm.at[wid, pl.ds(r0, rows_per_chunk), :],
                            ridx_v)
            for i in range(rows_per_chunk):
                for k in range(_IDXW // 16):
                    r = ridx_v[i, pl.ds(k * 16, 16)]
                    c = pidx_v[i, pl.ds(k * 16, 16)]
                    nidx_v[i, pl.ds(k * 16, 16)] = jnp.where(
                        r == c, lax.rem(r + 1, jnp.int32(v_size)), r)
            copies = []
            for i in range(rows_per_chunk):
                copies.append(pltpu.async_copy(
                    cb_hbm.at[pidx_v.at[i]],
                    pos_v.at[pl.ds(i * _IDXW, _IDXW)], sem))
            for i in range(rows_per_chunk):
                copies.append(pltpu.async_copy(
                    cb_hbm.at[nidx_v.at[i]],
                    neg_v.at[pl.ds(i * _IDXW, _IDXW)], sem))
            for cp in copies:
                cp.wait()

            def group_body(g, gcarry):
                al, ap, an = gcarry
                t0 = g * 16
                rows = t0 + lax.iota(jnp.int32, 16)

                dp2a = jnp.zeros((16,), jnp.float32)
                dp2b = jnp.zeros((16,), jnp.float32)
                dn2a = jnp.zeros((16,), jnp.float32)
                dn2b = jnp.zeros((16,), jnp.float32)

                def chan_octave(ci, ccarry):
                    dp2a, dp2b, dn2a, dn2b = ccarry
                    c0 = ci * _CUNROLL
                    for k in range(_CUNROLL):
                        c = c0 + k
                        col = jnp.full((16,), 0, jnp.int32) + c
                        f = f_v[c, pl.ds(t0, 16)]
                        p = plsc.load_gather(pos_v, [rows, col])
                        n = plsc.load_gather(neg_v, [rows, col])
                        dp = f - p
                        dn = f - n
                        if k % 2 == 0:
                            dp2a = dp2a + dp * dp
                            dn2a = dn2a + dn * dn
                        else:
                            dp2b = dp2b + dp * dp
                            dn2b = dn2b + dn * dn
                    return (dp2a, dp2b, dn2a, dn2b)

                dp2a, dp2b, dn2a, dn2b = lax.fori_loop(
                    0, C // _CUNROLL, chan_octave, (dp2a, dp2b, dn2a, dn2b))
                d_pos = _vsqrt(dp2a + dp2b)
                d_neg = _vsqrt(dn2a + dn2b)
                t = jnp.maximum(d_pos - d_neg + _MARGIN, 0.0)
                return (al + t, ap + d_pos, an + d_neg)

            return lax.fori_loop(0, groups, group_body,
                                 (acc_l, acc_p, acc_n))

        zero = jnp.zeros((16,), jnp.float32)
        acc_l, acc_p, acc_n = lax.fori_loop(0, n_chunk, chunk_body,
                                            (zero, zero, zero))
        res_v[0, :] = acc_l
        res_v[1, :] = acc_p
        res_v[2, :] = acc_n
        res_v[3, :] = jnp.zeros((16,), jnp.float32)
        pltpu.sync_copy(res_v, out_hbm.at[wid])

    return sc_kernel(student_features, codes3, rand3, codebook)


def kernel(student_features, teacher_codes, codebook):
    B, C, L = student_features.shape
    if teacher_codes.ndim == 3:
        teacher_codes = teacher_codes[0]
    V = codebook.shape[0]
    N = B * L
    rand = jax.random.randint(jax.random.key(42), (N,), 0, V)
    codes3 = teacher_codes.reshape(B, L // _IDXW, _IDXW).astype(jnp.int32)
    rand3 = rand.reshape(B, L // _IDXW, _IDXW).astype(jnp.int32)
    cb_adj = codebook - jnp.float32(_EPS)
    part = _sc_triplet(student_features, codes3, rand3, cb_adj, V)
    sums = part[:, :3, :].sum(axis=(0, 2))
    inv = jnp.float32(1.0 / N)
    return (sums[0] * inv, sums[1] * inv, sums[2] * inv)


# speedup vs baseline: 6.3610x; 2.5485x over previous
import functools

import jax
import jax.numpy as jnp
from jax import lax
from jax.experimental import pallas as pl
from jax.experimental.pallas import tpu as pltpu
from jax.experimental.pallas import tpu_sc as plsc

_MARGIN = 0.2
_EPS = 1e-6

_NC = 2
_NS = 16
_NW = _NC * _NS

_TCH = 512
_IDXW = 128
_CUNROLL = 8


def _vsqrt(s):
    i = lax.bitcast_convert_type(s, jnp.int32)
    y = lax.bitcast_convert_type(
        jnp.int32(0x5F3759DF) - lax.shift_right_logical(i, 1), jnp.float32)
    for _ in range(4):
        y = y * (1.5 - 0.5 * s * y * y)
    return s * y


@functools.partial(jax.jit, static_argnums=(4,))
def _sc_triplet(student_features, codes3, rand3, codebook, v_size):
    B, C, L = student_features.shape
    n_chunk = L // _TCH
    rows_per_chunk = _TCH // _IDXW
    groups = _TCH // 16

    mesh = plsc.VectorSubcoreMesh(core_axis_name="c", subcore_axis_name="s")

    @functools.partial(
        pl.kernel,
        mesh=mesh,
        compiler_params=pltpu.CompilerParams(
            needs_layout_passes=False, use_tc_tiling_on_sc=False),
        out_type=jax.ShapeDtypeStruct((_NW, 4, 16), jnp.float32),
        scratch_types=[
            pltpu.VMEM((C, _TCH), jnp.float32),
            pltpu.VMEM((rows_per_chunk, _IDXW), jnp.int32),
            pltpu.VMEM((rows_per_chunk, _IDXW), jnp.int32),
            pltpu.VMEM((rows_per_chunk, _IDXW), jnp.int32),
            pltpu.VMEM((_TCH, C), jnp.float32),
            pltpu.VMEM((_TCH, C), jnp.float32),
            pltpu.VMEM((4, 16), jnp.float32),
            pltpu.SemaphoreType.DMA,
        ],
    )
    def sc_kernel(sf_hbm, codes_hbm, rand_hbm, cb_hbm, out_hbm,
                  f_v, pidx_v, ridx_v, nidx_v, pos_v, neg_v, res_v, sem):
        cid = lax.axis_index("c")
        sid = lax.axis_index("s")
        wid = sid * _NC + cid

        def chunk_body(j, carry):
            acc_l, acc_p, acc_n = carry
            l0 = j * _TCH
            r0 = j * rows_per_chunk
            pltpu.sync_copy(sf_hbm.at[wid, :, pl.ds(l0, _TCH)], f_v)
            pltpu.sync_copy(codes_hbm.at[wid, pl.ds(r0, rows_per_chunk), :],
                            pidx_v)
            pltpu.sync_copy(rand_hbm.at[wid, pl.ds(r0, rows_per_chunk), :],
                            ridx_v)
            for i in range(rows_per_chunk):
                for k in range(_IDXW // 16):
                    r = ridx_v[i, pl.ds(k * 16, 16)]
                    c = pidx_v[i, pl.ds(k * 16, 16)]
                    nidx_v[i, pl.ds(k * 16, 16)] = jnp.where(
                        r == c, lax.rem(r + 1, jnp.int32(v_size)), r)
            copies = []
            for i in range(rows_per_chunk):
                copies.append(pltpu.async_copy(
                    cb_hbm.at[pidx_v.at[i]],
                    pos_v.at[pl.ds(i * _IDXW, _IDXW)], sem))
            for i in range(rows_per_chunk):
                copies.append(pltpu.async_copy(
                    cb_hbm.at[nidx_v.at[i]],
                    neg_v.at[pl.ds(i * _IDXW, _IDXW)], sem))
            for cp in copies:
                cp.wait()

            def group_body(g, gcarry):
                al, ap, an = gcarry
                t0 = g * 16
                lane = lax.iota(jnp.int32, 16)
                rows = t0 + lane

                dp2a = jnp.zeros((16,), jnp.float32)
                dp2b = jnp.zeros((16,), jnp.float32)
                dn2a = jnp.zeros((16,), jnp.float32)
                dn2b = jnp.zeros((16,), jnp.float32)

                def chan_octave(ci, ccarry):
                    dp2a, dp2b, dn2a, dn2b = ccarry
                    c0 = ci * _CUNROLL
                    for k in range(_CUNROLL):
                        col = lax.bitwise_xor(
                            jnp.full((16,), 0, jnp.int32) + (c0 + k), lane)
                        f = plsc.load_gather(f_v, [col, rows])
                        p = plsc.load_gather(pos_v, [rows, col])
                        n = plsc.load_gather(neg_v, [rows, col])
                        dp = f - p
                        dn = f - n
                        if k % 2 == 0:
                            dp2a = dp2a + dp * dp
                            dn2a = dn2a + dn * dn
                        else:
                            dp2b = dp2b + dp * dp
                            dn2b = dn2b + dn * dn
                    return (dp2a, dp2b, dn2a, dn2b)

                dp2a, dp2b, dn2a, dn2b = lax.fori_loop(
                    0, C // _CUNROLL, chan_octave, (dp2a, dp2b, dn2a, dn2b))
                d_pos = _vsqrt(dp2a + dp2b)
                d_neg = _vsqrt(dn2a + dn2b)
                t = jnp.maximum(d_pos - d_neg + _MARGIN, 0.0)
                return (al + t, ap + d_pos, an + d_neg)

            return lax.fori_loop(0, groups, group_body,
                                 (acc_l, acc_p, acc_n))

        zero = jnp.zeros((16,), jnp.float32)
        acc_l, acc_p, acc_n = lax.fori_loop(0, n_chunk, chunk_body,
                                            (zero, zero, zero))
        res_v[0, :] = acc_l
        res_v[1, :] = acc_p
        res_v[2, :] = acc_n
        res_v[3, :] = jnp.zeros((16,), jnp.float32)
        pltpu.sync_copy(res_v, out_hbm.at[wid])

    return sc_kernel(student_features, codes3, rand3, codebook)


def kernel(student_features, teacher_codes, codebook):
    B, C, L = student_features.shape
    if teacher_codes.ndim == 3:
        teacher_codes = teacher_codes[0]
    V = codebook.shape[0]
    N = B * L
    rand = jax.random.randint(jax.random.key(42), (N,), 0, V)
    codes3 = teacher_codes.reshape(B, L // _IDXW, _IDXW).astype(jnp.int32)
    rand3 = rand.reshape(B, L // _IDXW, _IDXW).astype(jnp.int32)
    cb_adj = codebook - jnp.float32(_EPS)
    part = _sc_triplet(student_features, codes3, rand3, cb_adj, V)
    sums = part[:, :3, :].sum(axis=(0, 2))
    inv = jnp.float32(1.0 / N)
    return (sums[0] * inv, sums[1] * inv, sums[2] * inv)


# speedup vs baseline: 9.0137x; 1.4170x over previous
import functools

import jax
import jax.numpy as jnp
from jax import lax
from jax.experimental import pallas as pl
from jax.experimental.pallas import tpu as pltpu
from jax.experimental.pallas import tpu_sc as plsc

_MARGIN = 0.2
_EPS = 1e-6

_NC = 2
_NS = 16
_NW = _NC * _NS

_TCH = 256
_IDXW = 128
_CUNROLL = 8


def _vsqrt(s):
    i = lax.bitcast_convert_type(s, jnp.int32)
    y = lax.bitcast_convert_type(
        jnp.int32(0x5F3759DF) - lax.shift_right_logical(i, 1), jnp.float32)
    for _ in range(4):
        y = y * (1.5 - 0.5 * s * y * y)
    return s * y


@functools.partial(jax.jit, static_argnums=(4,))
def _sc_triplet(student_features, codes3, rand3, codebook, v_size):
    B, C, L = student_features.shape
    n_chunk = L // _TCH
    rows_per_chunk = _TCH // _IDXW
    idx_rows = L // _IDXW
    groups = _TCH // 16

    mesh = plsc.VectorSubcoreMesh(core_axis_name="c", subcore_axis_name="s")

    @functools.partial(
        pl.kernel,
        mesh=mesh,
        compiler_params=pltpu.CompilerParams(
            needs_layout_passes=False, use_tc_tiling_on_sc=False),
        out_type=jax.ShapeDtypeStruct((_NW, 4, 16), jnp.float32),
        scratch_types=[
            pltpu.VMEM((2, C, _TCH), jnp.float32),
            pltpu.VMEM((2, _TCH, C), jnp.float32),
            pltpu.VMEM((2, _TCH, C), jnp.float32),
            pltpu.VMEM((idx_rows, _IDXW), jnp.int32),
            pltpu.VMEM((idx_rows, _IDXW), jnp.int32),
            pltpu.VMEM((4, 16), jnp.float32),
            pltpu.SemaphoreType.DMA,
            pltpu.SemaphoreType.DMA,
        ],
    )
    def sc_kernel(sf_hbm, codes_hbm, rand_hbm, cb_hbm, out_hbm,
                  f_v, pos_v, neg_v, pidx_v, nidx_v, res_v, sem0, sem1):
        cid = lax.axis_index("c")
        sid = lax.axis_index("s")
        wid = sid * _NC + cid
        sems = (sem0, sem1)

        pltpu.sync_copy(codes_hbm.at[wid], pidx_v)
        pltpu.sync_copy(rand_hbm.at[wid], nidx_v)

        def fix_body(i, _):
            for k in range(_IDXW // 16):
                r = nidx_v[i, pl.ds(k * 16, 16)]
                c = pidx_v[i, pl.ds(k * 16, 16)]
                nidx_v[i, pl.ds(k * 16, 16)] = jnp.where(
                    r == c, lax.rem(r + 1, jnp.int32(v_size)), r)
            return 0
        lax.fori_loop(0, idx_rows, fix_body, 0)

        def issue_chunk(j, q):
            l0 = j * _TCH
            r0 = j * rows_per_chunk
            pltpu.async_copy(
                sf_hbm.at[wid, :, pl.ds(l0, _TCH)], f_v.at[q], sems[q])
            for i in range(rows_per_chunk):
                pltpu.async_copy(
                    cb_hbm.at[pidx_v.at[r0 + i]],
                    pos_v.at[q, pl.ds(i * _IDXW, _IDXW)], sems[q])
                pltpu.async_copy(
                    cb_hbm.at[nidx_v.at[r0 + i]],
                    neg_v.at[q, pl.ds(i * _IDXW, _IDXW)], sems[q])

        def wait_chunk(q):
            pltpu.make_async_copy(
                sf_hbm.at[0, :, pl.ds(0, _TCH)], f_v.at[q], sems[q]).wait()
            for i in range(rows_per_chunk):
                pltpu.make_async_copy(
                    cb_hbm.at[pidx_v.at[i]],
                    pos_v.at[q, pl.ds(i * _IDXW, _IDXW)], sems[q]).wait()
                pltpu.make_async_copy(
                    cb_hbm.at[nidx_v.at[i]],
                    neg_v.at[q, pl.ds(i * _IDXW, _IDXW)], sems[q]).wait()

        def compute_chunk(q, carry):
            fq, pq, nq = f_v.at[q], pos_v.at[q], neg_v.at[q]
            def group_body(g, gcarry):
                al, ap, an = gcarry
                t0 = g * 16
                lane = lax.iota(jnp.int32, 16)
                rows = t0 + lane
                dp2a = jnp.zeros((16,), jnp.float32)
                dp2b = jnp.zeros((16,), jnp.float32)
                dn2a = jnp.zeros((16,), jnp.float32)
                dn2b = jnp.zeros((16,), jnp.float32)

                def chan_octave(ci, ccarry):
                    dp2a, dp2b, dn2a, dn2b = ccarry
                    c0 = ci * _CUNROLL
                    for k in range(_CUNROLL):
                        col = lax.bitwise_xor(
                            jnp.full((16,), 0, jnp.int32) + (c0 + k), lane)
                        f = plsc.load_gather(fq, [col, rows])
                        p = plsc.load_gather(pq, [rows, col])
                        n = plsc.load_gather(nq, [rows, col])
                        dp = f - p
                        dn = f - n
                        if k % 2 == 0:
                            dp2a = dp2a + dp * dp
                            dn2a = dn2a + dn * dn
                        else:
                            dp2b = dp2b + dp * dp
                            dn2b = dn2b + dn * dn
                    return (dp2a, dp2b, dn2a, dn2b)

                dp2a, dp2b, dn2a, dn2b = lax.fori_loop(
                    0, C // _CUNROLL, chan_octave, (dp2a, dp2b, dn2a, dn2b))
                d_pos = _vsqrt(dp2a + dp2b)
                d_neg = _vsqrt(dn2a + dn2b)
                t = jnp.maximum(d_pos - d_neg + _MARGIN, 0.0)
                return (al + t, ap + d_pos, an + d_neg)

            return lax.fori_loop(0, groups, group_body, carry)

        issue_chunk(0, 0)

        def pair_body(m, carry):
            j0 = 2 * m

            @pl.when(j0 + 1 < n_chunk)
            def _():
                issue_chunk(j0 + 1, 1)
            wait_chunk(0)
            carry0 = compute_chunk(0, carry)

            @pl.when(j0 + 2 < n_chunk)
            def _():
                issue_chunk(j0 + 2, 0)
            wait_chunk(1)
            return compute_chunk(1, carry0)

        zero = jnp.zeros((16,), jnp.float32)
        acc_l, acc_p, acc_n = lax.fori_loop(0, n_chunk // 2, pair_body,
                                            (zero, zero, zero))
        res_v[0, :] = acc_l
        res_v[1, :] = acc_p
        res_v[2, :] = acc_n
        res_v[3, :] = jnp.zeros((16,), jnp.float32)
        pltpu.sync_copy(res_v, out_hbm.at[wid])

    return sc_kernel(student_features, codes3, rand3, codebook)


def kernel(student_features, teacher_codes, codebook):
    B, C, L = student_features.shape
    if teacher_codes.ndim == 3:
        teacher_codes = teacher_codes[0]
    V = codebook.shape[0]
    N = B * L
    rand = jax.random.randint(jax.random.key(42), (N,), 0, V)
    codes3 = teacher_codes.reshape(B, L // _IDXW, _IDXW).astype(jnp.int32)
    rand3 = rand.reshape(B, L // _IDXW, _IDXW).astype(jnp.int32)
    cb_adj = codebook - jnp.float32(_EPS)
    part = _sc_triplet(student_features, codes3, rand3, cb_adj, V)
    sums = part[:, :3, :].sum(axis=(0, 2))
    inv = jnp.float32(1.0 / N)
    return (sums[0] * inv, sums[1] * inv, sums[2] * inv)
